# baseline (speedup 1.0000x reference)
import functools

import jax
import jax.numpy as jnp
from jax import lax
from jax.experimental import pallas as pl
from jax.experimental.pallas import tpu as pltpu
from jax.experimental.pallas import tpu_sc as plsc

_B, _T, _D = 4, 2048, 768
_DW = _D // 2
_N = _B * _T
_E = 8
_EP = 128
_BM = 512
_NBLK = _N // _BM
_BMC = 256
_NS = _N * 2 + _E * _BMC
_NT = _NS // _BMC
_NEG = -1e30

_NW = 32
_TPW = _N // _NW
_CH = 64
_NCH = _TPW // _CH

_MHI = -65536
_RND = 0x8000


def _route_block(x_ref, gw_ref, gb_ref, xb_ref, meta_ref, metaf_ref, pg_ref,
                 counts_ref, carry_ref):
    i = pl.program_id(0)

    @pl.when(i == 0)
    def _():
        carry_ref[...] = jnp.zeros((1, _E), jnp.float32)

    x = x_ref[...]
    xb_ref[...] = x.astype(jnp.bfloat16)
    logits = lax.dot_general(x, gw_ref[...], (((1,), (1,)), ((), ())),
                             preferred_element_type=jnp.float32) + gb_ref[...]
    eidx = lax.broadcasted_iota(jnp.int32, logits.shape, 1)
    m1 = jnp.max(logits, axis=1, keepdims=True)
    a1 = jnp.min(jnp.where(logits >= m1, eidx, _EP), axis=1, keepdims=True)
    mask1 = eidx == a1
    l2 = jnp.where(mask1, _NEG, logits)
    m2 = jnp.max(l2, axis=1, keepdims=True)
    a2 = jnp.min(jnp.where(l2 >= m2, eidx, _EP), axis=1, keepdims=True)
    mask2 = eidx == a2
    w1 = jax.nn.sigmoid(m1)
    w2 = jax.nn.sigmoid(m2)

    m1e = mask1[:, :_E]
    m2e = mask2[:, :_E]
    cnt = m1e.astype(jnp.float32) + m2e.astype(jnp.float32)
    r_io = lax.broadcasted_iota(jnp.int32, (_BM, _BM), 0)
    c_io = lax.broadcasted_iota(jnp.int32, (_BM, _BM), 1)
    ltri = (r_io > c_io).astype(jnp.float32)
    excl = lax.dot_general(ltri, cnt, (((1,), (0,)), ((), ())),
                           preferred_element_type=jnp.float32)
    rankf = excl + carry_ref[...]
    r1 = jnp.sum(jnp.where(m1e, rankf, 0.0), axis=1, keepdims=True)
    r2 = jnp.sum(jnp.where(m2e, rankf, 0.0), axis=1, keepdims=True)
    carry_ref[...] = carry_ref[...] + jnp.sum(cnt, axis=0, keepdims=True)

    zi = jnp.zeros((_BM, 2), jnp.int32)
    meta_ref[...] = jnp.concatenate(
        [a1, a2, r1.astype(jnp.int32), r2.astype(jnp.int32), zi, zi], axis=1)
    zf = jnp.zeros((_BM, 2), jnp.float32)
    metaf_ref[...] = jnp.concatenate([w1, w2, zf, zf, zf], axis=1)

    @pl.when(i == _NBLK - 1)
    def _():
        total = carry_ref[...]
        aligned = jnp.ceil(total / _BMC) * _BMC
        e_r = lax.broadcasted_iota(jnp.int32, (_E, _E), 0)
        e_c = lax.broadcasted_iota(jnp.int32, (_E, _E), 1)
        before = (e_r < e_c).astype(jnp.float32)
        pg = lax.dot_general(aligned, before, (((1,), (0,)), ((), ())),
                             preferred_element_type=jnp.float32)
        pg_ref[...] = jnp.concatenate(
            [pg.astype(jnp.int32), jnp.zeros((1, 16 - _E), jnp.int32)],
            axis=1)
        counts_ref[...] = total.astype(jnp.int32)


def _route(x, gw_p, gb_p):
    return pl.pallas_call(
        _route_block,
        grid=(_NBLK,),
        in_specs=[
            pl.BlockSpec((_BM, _D), lambda i: (i, 0)),
            pl.BlockSpec((_EP, _D), lambda i: (0, 0)),
            pl.BlockSpec((1, _EP), lambda i: (0, 0)),
        ],
        out_specs=[
            pl.BlockSpec((_BM, _D), lambda i: (i, 0)),
            pl.BlockSpec((_BM, 8), lambda i: (i, 0)),
            pl.BlockSpec((_BM, 8), lambda i: (i, 0)),
            pl.BlockSpec((1, 16), lambda i: (0, 0)),
            pl.BlockSpec((1, _E), lambda i: (0, 0)),
        ],
        out_shape=[
            jax.ShapeDtypeStruct((_N, _D), jnp.bfloat16),
            jax.ShapeDtypeStruct((_N, 8), jnp.int32),
            jax.ShapeDtypeStruct((_N, 8), jnp.float32),
            jax.ShapeDtypeStruct((1, 16), jnp.int32),
            jax.ShapeDtypeStruct((1, _E), jnp.int32),
        ],
        scratch_shapes=[pltpu.VMEM((1, _E), jnp.float32)],
        compiler_params=pltpu.CompilerParams(
            dimension_semantics=("arbitrary",),
        ),
    )(x, gw_p, gb_p)


def _pos_block(meta_ref, metaf_ref, pg_ref, pos1_ref, pos2_ref,
               wr1_ref, wr2_ref):
    m = meta_ref[...]
    pg = pg_ref[...]
    ei = lax.broadcasted_iota(jnp.int32, (_N, 16), 1)
    s1 = jnp.sum(jnp.where(ei == m[:, 0:1], pg, 0), axis=1)
    s2 = jnp.sum(jnp.where(ei == m[:, 1:2], pg, 0), axis=1)
    pos1_ref[...] = s1 + m[:, 2]
    pos2_ref[...] = s2 + m[:, 3]
    mf = metaf_ref[...]
    wr1_ref[...] = jnp.broadcast_to(mf[:, 0:1], (_N, 16))
    wr2_ref[...] = jnp.broadcast_to(mf[:, 1:2], (_N, 16))


def _positions(meta, metaf, pg16):
    return pl.pallas_call(
        _pos_block,
        out_shape=[
            jax.ShapeDtypeStruct((_N,), jnp.int32),
            jax.ShapeDtypeStruct((_N,), jnp.int32),
            jax.ShapeDtypeStruct((_N, 16), jnp.float32),
            jax.ShapeDtypeStruct((_N, 16), jnp.float32),
        ],
    )(meta, metaf, pg16)


def _sc_wid():
    return lax.axis_index("s") * 2 + lax.axis_index("c")


@functools.cache
def _build_dispatch():
    mesh = plsc.VectorSubcoreMesh(core_axis_name="c", subcore_axis_name="s")

    @functools.partial(
        pl.kernel, mesh=mesh,
        out_type=jax.ShapeDtypeStruct((_NS, _DW), jnp.int32),
        scratch_types=[
            pltpu.VMEM((_NCH, _CH), jnp.int32),
            pltpu.VMEM((_NCH, _CH), jnp.int32),
            pltpu.VMEM((_CH, _DW), jnp.int32),
            pltpu.VMEM((_CH, _DW), jnp.int32),
            pltpu.SemaphoreType.DMA,
            pltpu.SemaphoreType.DMA,
            pltpu.SemaphoreType.DMA,
            pltpu.SemaphoreType.DMA,
            pltpu.SemaphoreType.DMA,
            pltpu.SemaphoreType.DMA,
        ],
    )
    def _dispatch(xb_hbm, pos1_hbm, pos2_hbm, xs_hbm,
                  p1a, p2a, v0, v1, l0, l1, sa0, sb0, sa1, sb1):
        base = _sc_wid() * _TPW
        row0 = _sc_wid() * _NCH
        pltpu.sync_copy(pos1_hbm.at[pl.ds(row0, _NCH)], p1a)
        pltpu.sync_copy(pos2_hbm.at[pl.ds(row0, _NCH)], p2a)
        bufs = (v0, v1)
        lsems = (l0, l1)
        ssems = ((sa0, sb0), (sa1, sb1))
        loads = [None] * _NCH
        scat = [None] * _NCH

        def start_load(ci):
            loads[ci] = pltpu.async_copy(
                xb_hbm.at[pl.ds(base + ci * _CH, _CH)], bufs[ci % 2],
                lsems[ci % 2])

        start_load(0)
        for ci in range(_NCH):
            if ci >= 1:
                ca, cb = scat[ci - 1]
                ca.wait()
                cb.wait()
            if ci + 1 < _NCH:
                start_load(ci + 1)
            loads[ci].wait()
            sa, sb = ssems[ci % 2]
            ca = pltpu.async_copy(bufs[ci % 2], xs_hbm.at[p1a.at[ci]], sa)
            cb = pltpu.async_copy(bufs[ci % 2], xs_hbm.at[p2a.at[ci]], sb)
            scat[ci] = (ca, cb)
        ca, cb = scat[_NCH - 1]
        ca.wait()
        cb.wait()

    return _dispatch


def _gmm_block(te_ref, xs_ref, w_ref, b_ref, ys_ref):
    acc = lax.dot_general(xs_ref[...], w_ref[0], (((1,), (1,)), ((), ())),
                          preferred_element_type=jnp.float32)
    ys_ref[...] = (acc + b_ref[0]).astype(jnp.bfloat16)


def _gmm(xs_bf, w_bf16, b3, tile_expert):
    return pl.pallas_call(
        _gmm_block,
        grid_spec=pltpu.PrefetchScalarGridSpec(
            num_scalar_prefetch=1,
            grid=(_NT,),
            in_specs=[
                pl.BlockSpec((_BMC, _D), lambda i, te: (i, 0)),
                pl.BlockSpec((1, _D, _D), lambda i, te: (te[i], 0, 0)),
                pl.BlockSpec((1, 1, _D), lambda i, te: (te[i], 0, 0)),
            ],
            out_specs=pl.BlockSpec((_BMC, _D), lambda i, te: (i, 0)),
        ),
        out_shape=jax.ShapeDtypeStruct((_NS, _D), jnp.bfloat16),
        compiler_params=pltpu.CompilerParams(
            dimension_semantics=("arbitrary",),
        ),
    )(tile_expert, xs_bf, w_bf16, b3)


@functools.cache
def _build_combine():
    mesh = plsc.VectorSubcoreMesh(core_axis_name="c", subcore_axis_name="s")

    @functools.partial(
        pl.kernel, mesh=mesh,
        out_type=jax.ShapeDtypeStruct((_N, _DW), jnp.int32),
        scratch_types=[
            pltpu.VMEM((_NCH, _CH), jnp.int32),
            pltpu.VMEM((_NCH, _CH), jnp.int32),
            pltpu.VMEM((_CH, 16), jnp.float32),
            pltpu.VMEM((_CH, 16), jnp.float32),
            pltpu.VMEM((_CH, _DW), jnp.int32),
            pltpu.VMEM((_CH, _DW), jnp.int32),
            pltpu.SemaphoreType.DMA,
            pltpu.SemaphoreType.DMA,
        ],
    )
    def _combine(ys_hbm, pos1_hbm, pos2_hbm, wr1_hbm, wr2_hbm, y_hbm,
                 p1a, p2a, wb1, wb2, b1, b2, s1, s2):
        base = _sc_wid() * _TPW
        row0 = _sc_wid() * _NCH
        pltpu.sync_copy(pos1_hbm.at[pl.ds(row0, _NCH)], p1a)
        pltpu.sync_copy(pos2_hbm.at[pl.ds(row0, _NCH)], p2a)
        for ci in range(_NCH):
            t0 = base + ci * _CH
            pltpu.sync_copy(wr1_hbm.at[pl.ds(t0, _CH)], wb1)
            pltpu.sync_copy(wr2_hbm.at[pl.ds(t0, _CH)], wb2)
            cp1 = pltpu.async_copy(ys_hbm.at[p1a.at[ci]], b1, s1)
            cp2 = pltpu.async_copy(ys_hbm.at[p2a.at[ci]], b2, s2)
            cp1.wait()
            cp2.wait()

            def loop(t, _):
                w1 = wb1[t, :]
                w2 = wb2[t, :]
                for v in range(_DW // 16):
                    sl = pl.ds(v * 16, 16)
                    a = b1[t, sl]
                    b = b2[t, sl]
                    _f32 = functools.partial(lax.bitcast_convert_type,
                                             new_dtype=jnp.float32)
                    _i32 = functools.partial(lax.bitcast_convert_type,
                                             new_dtype=jnp.int32)
                    lo = (w1 * _f32(lax.shift_left(a, 16))
                          + w2 * _f32(lax.shift_left(b, 16)))
                    hi = w1 * _f32(a & _MHI) + w2 * _f32(b & _MHI)
                    lob = lax.shift_right_logical(_i32(lo) + _RND, 16)
                    hib = (_i32(hi) + _RND) & _MHI
                    b1[t, sl] = lob | hib
                return 0

            lax.fori_loop(0, _CH, loop, 0)
            pltpu.sync_copy(b1, y_hbm.at[pl.ds(t0, _CH)])

    return _combine


def kernel(inputs, gate_W, gate_b, expert_W, expert_b):
    x = inputs.reshape(_N, _D)
    gw_p = jnp.zeros((_EP, _D), jnp.float32).at[:_E].set(gate_W)
    gb_p = jnp.full((1, _EP), _NEG, jnp.float32).at[0, :_E].set(gate_b)
    w_bf16 = expert_W.astype(jnp.bfloat16)
    b3 = expert_b.reshape(_E, 1, _D)

    xb16, meta, metaf, pg16, counts = _route(x, gw_p, gb_p)
    pos1, pos2, wr1, wr2 = _positions(meta, metaf, pg16)

    aligned = (counts[0] + (_BMC - 1)) // _BMC * _BMC
    ends = jnp.cumsum(aligned)
    tile_starts = jnp.arange(_NT, dtype=jnp.int32) * _BMC
    tile_expert = jnp.sum(
        (tile_starts[:, None] >= ends[None, :_E - 1]).astype(jnp.int32),
        axis=1)

    xb_i32 = lax.bitcast_convert_type(
        xb16.reshape(_N, _DW, 2), jnp.int32)
    pos1_2d = pos1.reshape(_N // _CH, _CH)
    pos2_2d = pos2.reshape(_N // _CH, _CH)

    xs_i32 = _build_dispatch()(xb_i32, pos1_2d, pos2_2d)
    xs_bf = lax.bitcast_convert_type(xs_i32, jnp.bfloat16).reshape(_NS, _D)
    ys_bf = _gmm(xs_bf, w_bf16, b3, tile_expert)
    ys_i32 = lax.bitcast_convert_type(
        ys_bf.reshape(_NS, _DW, 2), jnp.int32)
    y_i32 = _build_combine()(ys_i32, pos1_2d, pos2_2d, wr1, wr2)
    y = lax.bitcast_convert_type(y_i32, jnp.bfloat16).reshape(_N, _D)
    return y.astype(jnp.float32).reshape(_B, _T, _D)


# speedup vs baseline: 4.5562x; 4.5562x over previous
import functools

import jax
import jax.numpy as jnp
from jax import lax
from jax.experimental import pallas as pl
from jax.experimental.pallas import tpu as pltpu
from jax.experimental.pallas import tpu_sc as plsc

_B, _T, _D = 4, 2048, 768
_DW = _D // 2
_N = _B * _T
_E = 8
_EP = 128
_BM = 512
_NBLK = _N // _BM
_BMC = 256
_NS = _N * 2 + _E * _BMC
_NT = _NS // _BMC
_NEG = -1e30

_NW = 32
_TPW = _N // _NW
_CH = 64
_NCH = _TPW // _CH

_MHI = -65536
_RND = 0x8000


def _route_block(x_ref, gw_ref, gb_ref, xb_ref, meta_ref, metaf_ref, pg_ref,
                 counts_ref, carry_ref):
    i = pl.program_id(0)

    @pl.when(i == 0)
    def _():
        carry_ref[...] = jnp.zeros((1, _E), jnp.float32)

    x = x_ref[...]
    bits = lax.bitcast_convert_type(x, jnp.int32)
    rb = bits + 0x7FFF + (lax.shift_right_logical(bits, 16) & 1)
    xb_ref[...] = (lax.shift_right_logical(rb[:, :_DW], 16)
                   | (rb[:, _DW:] & _MHI))
    logits = lax.dot_general(x, gw_ref[...], (((1,), (1,)), ((), ())),
                             preferred_element_type=jnp.float32) + gb_ref[...]
    eidx = lax.broadcasted_iota(jnp.int32, logits.shape, 1)
    m1 = jnp.max(logits, axis=1, keepdims=True)
    a1 = jnp.min(jnp.where(logits >= m1, eidx, _EP), axis=1, keepdims=True)
    mask1 = eidx == a1
    l2 = jnp.where(mask1, _NEG, logits)
    m2 = jnp.max(l2, axis=1, keepdims=True)
    a2 = jnp.min(jnp.where(l2 >= m2, eidx, _EP), axis=1, keepdims=True)
    mask2 = eidx == a2
    w1 = jax.nn.sigmoid(m1)
    w2 = jax.nn.sigmoid(m2)

    m1e = mask1[:, :_E]
    m2e = mask2[:, :_E]
    cnt = m1e.astype(jnp.float32) + m2e.astype(jnp.float32)
    r_io = lax.broadcasted_iota(jnp.int32, (_BM, _BM), 0)
    c_io = lax.broadcasted_iota(jnp.int32, (_BM, _BM), 1)
    ltri = (r_io > c_io).astype(jnp.float32)
    excl = lax.dot_general(ltri, cnt, (((1,), (0,)), ((), ())),
                           preferred_element_type=jnp.float32)
    rankf = excl + carry_ref[...]
    r1 = jnp.sum(jnp.where(m1e, rankf, 0.0), axis=1, keepdims=True)
    r2 = jnp.sum(jnp.where(m2e, rankf, 0.0), axis=1, keepdims=True)
    carry_ref[...] = carry_ref[...] + jnp.sum(cnt, axis=0, keepdims=True)

    zi = jnp.zeros((_BM, 2), jnp.int32)
    meta_ref[...] = jnp.concatenate(
        [a1, a2, r1.astype(jnp.int32), r2.astype(jnp.int32), zi, zi], axis=1)
    zf = jnp.zeros((_BM, 2), jnp.float32)
    metaf_ref[...] = jnp.concatenate([w1, w2, zf, zf, zf], axis=1)

    @pl.when(i == _NBLK - 1)
    def _():
        total = carry_ref[...]
        aligned = jnp.ceil(total / _BMC) * _BMC
        e_r = lax.broadcasted_iota(jnp.int32, (_E, _E), 0)
        e_c = lax.broadcasted_iota(jnp.int32, (_E, _E), 1)
        before = (e_r < e_c).astype(jnp.float32)
        pg = lax.dot_general(aligned, before, (((1,), (0,)), ((), ())),
                             preferred_element_type=jnp.float32)
        pg_ref[...] = jnp.concatenate(
            [pg.astype(jnp.int32), jnp.zeros((1, 16 - _E), jnp.int32)],
            axis=1)
        counts_ref[...] = total.astype(jnp.int32)


def _route(x, gw_p, gb_p):
    return pl.pallas_call(
        _route_block,
        grid=(_NBLK,),
        in_specs=[
            pl.BlockSpec((_BM, _D), lambda i: (i, 0)),
            pl.BlockSpec((_EP, _D), lambda i: (0, 0)),
            pl.BlockSpec((1, _EP), lambda i: (0, 0)),
        ],
        out_specs=[
            pl.BlockSpec((_BM, _DW), lambda i: (i, 0)),
            pl.BlockSpec((_BM, 8), lambda i: (i, 0)),
            pl.BlockSpec((_BM, 8), lambda i: (i, 0)),
            pl.BlockSpec((1, 16), lambda i: (0, 0)),
            pl.BlockSpec((1, _E), lambda i: (0, 0)),
        ],
        out_shape=[
            jax.ShapeDtypeStruct((_N, _DW), jnp.int32),
            jax.ShapeDtypeStruct((_N, 8), jnp.int32),
            jax.ShapeDtypeStruct((_N, 8), jnp.float32),
            jax.ShapeDtypeStruct((1, 16), jnp.int32),
            jax.ShapeDtypeStruct((1, _E), jnp.int32),
        ],
        scratch_shapes=[pltpu.VMEM((1, _E), jnp.float32)],
        compiler_params=pltpu.CompilerParams(
            dimension_semantics=("arbitrary",),
        ),
    )(x, gw_p, gb_p)


def _pos_block(meta_ref, metaf_ref, pg_ref, pos1_ref, pos2_ref,
               wr1_ref, wr2_ref):
    m = meta_ref[...]
    pg = pg_ref[...]
    ei = lax.broadcasted_iota(jnp.int32, (_N, 16), 1)
    s1 = jnp.sum(jnp.where(ei == m[:, 0:1], pg, 0), axis=1)
    s2 = jnp.sum(jnp.where(ei == m[:, 1:2], pg, 0), axis=1)
    pos1_ref[...] = s1 + m[:, 2]
    pos2_ref[...] = s2 + m[:, 3]
    mf = metaf_ref[...]
    wr1_ref[...] = jnp.broadcast_to(mf[:, 0:1], (_N, 16))
    wr2_ref[...] = jnp.broadcast_to(mf[:, 1:2], (_N, 16))


def _positions(meta, metaf, pg16):
    return pl.pallas_call(
        _pos_block,
        out_shape=[
            jax.ShapeDtypeStruct((_N,), jnp.int32),
            jax.ShapeDtypeStruct((_N,), jnp.int32),
            jax.ShapeDtypeStruct((_N, 16), jnp.float32),
            jax.ShapeDtypeStruct((_N, 16), jnp.float32),
        ],
    )(meta, metaf, pg16)


def _sc_wid():
    return lax.axis_index("s") * 2 + lax.axis_index("c")


@functools.cache
def _build_dispatch():
    mesh = plsc.VectorSubcoreMesh(core_axis_name="c", subcore_axis_name="s")

    @functools.partial(
        pl.kernel, mesh=mesh,
        out_type=jax.ShapeDtypeStruct((_NS, _DW), jnp.int32),
        scratch_types=[
            pltpu.VMEM((_NCH, _CH), jnp.int32),
            pltpu.VMEM((_NCH, _CH), jnp.int32),
            pltpu.VMEM((_CH, _DW), jnp.int32),
            pltpu.VMEM((_CH, _DW), jnp.int32),
            pltpu.SemaphoreType.DMA,
            pltpu.SemaphoreType.DMA,
            pltpu.SemaphoreType.DMA,
            pltpu.SemaphoreType.DMA,
            pltpu.SemaphoreType.DMA,
            pltpu.SemaphoreType.DMA,
        ],
    )
    def _dispatch(xb_hbm, pos1_hbm, pos2_hbm, xs_hbm,
                  p1a, p2a, v0, v1, l0, l1, sa0, sb0, sa1, sb1):
        base = _sc_wid() * _TPW
        row0 = _sc_wid() * _NCH
        pltpu.sync_copy(pos1_hbm.at[pl.ds(row0, _NCH)], p1a)
        pltpu.sync_copy(pos2_hbm.at[pl.ds(row0, _NCH)], p2a)
        bufs = (v0, v1)
        lsems = (l0, l1)
        ssems = ((sa0, sb0), (sa1, sb1))
        loads = [None] * _NCH
        scat = [None] * _NCH

        def start_load(ci):
            loads[ci] = pltpu.async_copy(
                xb_hbm.at[pl.ds(base + ci * _CH, _CH)], bufs[ci % 2],
                lsems[ci % 2])

        start_load(0)
        for ci in range(_NCH):
            if ci >= 1:
                ca, cb = scat[ci - 1]
                ca.wait()
                cb.wait()
            if ci + 1 < _NCH:
                start_load(ci + 1)
            loads[ci].wait()
            sa, sb = ssems[ci % 2]
            ca = pltpu.async_copy(bufs[ci % 2], xs_hbm.at[p1a.at[ci]], sa)
            cb = pltpu.async_copy(bufs[ci % 2], xs_hbm.at[p2a.at[ci]], sb)
            scat[ci] = (ca, cb)
        ca, cb = scat[_NCH - 1]
        ca.wait()
        cb.wait()

    return _dispatch


def _unpack_f32(w):
    lo = lax.bitcast_convert_type(lax.shift_left(w, 16), jnp.float32)
    hi = lax.bitcast_convert_type(w & _MHI, jnp.float32)
    return jnp.concatenate([lo, hi], axis=1)


def _gmm_block(te_ref, xs_ref, w_ref, b_ref, ys_ref):
    xb = _unpack_f32(xs_ref[...]).astype(jnp.bfloat16)
    acc = lax.dot_general(xb, w_ref[0], (((1,), (1,)), ((), ())),
                          preferred_element_type=jnp.float32)
    acc = acc + b_ref[0]
    bits = lax.bitcast_convert_type(acc, jnp.int32)
    rb = bits + 0x7FFF + (lax.shift_right_logical(bits, 16) & 1)
    ys_ref[...] = (lax.shift_right_logical(rb[:, :_DW], 16)
                   | (rb[:, _DW:] & _MHI))


def _gmm(xs_i32, w_bf16, b3, tile_expert):
    return pl.pallas_call(
        _gmm_block,
        grid_spec=pltpu.PrefetchScalarGridSpec(
            num_scalar_prefetch=1,
            grid=(_NT,),
            in_specs=[
                pl.BlockSpec((_BMC, _DW), lambda i, te: (i, 0)),
                pl.BlockSpec((1, _D, _D), lambda i, te: (te[i], 0, 0)),
                pl.BlockSpec((1, 1, _D), lambda i, te: (te[i], 0, 0)),
            ],
            out_specs=pl.BlockSpec((_BMC, _DW), lambda i, te: (i, 0)),
        ),
        out_shape=jax.ShapeDtypeStruct((_NS, _DW), jnp.int32),
        compiler_params=pltpu.CompilerParams(
            dimension_semantics=("arbitrary",),
        ),
    )(tile_expert, xs_i32, w_bf16, b3)


def _unpack_block(yi_ref, o_ref):
    o_ref[...] = _unpack_f32(yi_ref[...])


def _unpack(y_i32):
    return pl.pallas_call(
        _unpack_block,
        grid=(_NBLK,),
        in_specs=[pl.BlockSpec((_BM, _DW), lambda i: (i, 0))],
        out_specs=pl.BlockSpec((_BM, _D), lambda i: (i, 0)),
        out_shape=jax.ShapeDtypeStruct((_N, _D), jnp.float32),
        compiler_params=pltpu.CompilerParams(
            dimension_semantics=("arbitrary",),
        ),
    )(y_i32)


@functools.cache
def _build_combine():
    mesh = plsc.VectorSubcoreMesh(core_axis_name="c", subcore_axis_name="s")

    @functools.partial(
        pl.kernel, mesh=mesh,
        out_type=jax.ShapeDtypeStruct((_N, _DW), jnp.int32),
        scratch_types=[
            pltpu.VMEM((_NCH, _CH), jnp.int32),
            pltpu.VMEM((_NCH, _CH), jnp.int32),
            pltpu.VMEM((_CH, 16), jnp.float32),
            pltpu.VMEM((_CH, 16), jnp.float32),
            pltpu.VMEM((_CH, _DW), jnp.int32),
            pltpu.VMEM((_CH, _DW), jnp.int32),
            pltpu.SemaphoreType.DMA,
            pltpu.SemaphoreType.DMA,
        ],
    )
    def _combine(ys_hbm, pos1_hbm, pos2_hbm, wr1_hbm, wr2_hbm, y_hbm,
                 p1a, p2a, wb1, wb2, b1, b2, s1, s2):
        base = _sc_wid() * _TPW
        row0 = _sc_wid() * _NCH
        pltpu.sync_copy(pos1_hbm.at[pl.ds(row0, _NCH)], p1a)
        pltpu.sync_copy(pos2_hbm.at[pl.ds(row0, _NCH)], p2a)
        for ci in range(_NCH):
            t0 = base + ci * _CH
            pltpu.sync_copy(wr1_hbm.at[pl.ds(t0, _CH)], wb1)
            pltpu.sync_copy(wr2_hbm.at[pl.ds(t0, _CH)], wb2)
            cp1 = pltpu.async_copy(ys_hbm.at[p1a.at[ci]], b1, s1)
            cp2 = pltpu.async_copy(ys_hbm.at[p2a.at[ci]], b2, s2)
            cp1.wait()
            cp2.wait()

            def loop(t, _):
                w1 = wb1[t, :]
                w2 = wb2[t, :]
                for v in range(_DW // 16):
                    sl = pl.ds(v * 16, 16)
                    a = b1[t, sl]
                    b = b2[t, sl]
                    _f32 = functools.partial(lax.bitcast_convert_type,
                                             new_dtype=jnp.float32)
                    _i32 = functools.partial(lax.bitcast_convert_type,
                                             new_dtype=jnp.int32)
                    lo = (w1 * _f32(lax.shift_left(a, 16))
                          + w2 * _f32(lax.shift_left(b, 16)))
                    hi = w1 * _f32(a & _MHI) + w2 * _f32(b & _MHI)
                    lob = lax.shift_right_logical(_i32(lo) + _RND, 16)
                    hib = (_i32(hi) + _RND) & _MHI
                    b1[t, sl] = lob | hib
                return 0

            lax.fori_loop(0, _CH, loop, 0)
            pltpu.sync_copy(b1, y_hbm.at[pl.ds(t0, _CH)])

    return _combine


def kernel(inputs, gate_W, gate_b, expert_W, expert_b):
    x = inputs.reshape(_N, _D)
    gw_p = jnp.zeros((_EP, _D), jnp.float32).at[:_E].set(gate_W)
    gb_p = jnp.full((1, _EP), _NEG, jnp.float32).at[0, :_E].set(gate_b)
    w_bf16 = expert_W.astype(jnp.bfloat16)
    b3 = expert_b.reshape(_E, 1, _D)

    xb_i32, meta, metaf, pg16, counts = _route(x, gw_p, gb_p)
    pos1, pos2, wr1, wr2 = _positions(meta, metaf, pg16)

    aligned = (counts[0] + (_BMC - 1)) // _BMC * _BMC
    ends = jnp.cumsum(aligned)
    tile_starts = jnp.arange(_NT, dtype=jnp.int32) * _BMC
    tile_expert = jnp.sum(
        (tile_starts[:, None] >= ends[None, :_E - 1]).astype(jnp.int32),
        axis=1)

    pos1_2d = pos1.reshape(_N // _CH, _CH)
    pos2_2d = pos2.reshape(_N // _CH, _CH)

    xs_i32 = _build_dispatch()(xb_i32, pos1_2d, pos2_2d)
    ys_i32 = _gmm(xs_i32, w_bf16, b3, tile_expert)
    y_i32 = _build_combine()(ys_i32, pos1_2d, pos2_2d, wr1, wr2)
    return _unpack(y_i32).reshape(_B, _T, _D)


# speedup vs baseline: 5.2704x; 1.1567x over previous
import functools

import jax
import jax.numpy as jnp
from jax import lax
from jax.experimental import pallas as pl
from jax.experimental.pallas import tpu as pltpu
from jax.experimental.pallas import tpu_sc as plsc

_B, _T, _D = 4, 2048, 768
_DW = _D // 2
_N = _B * _T
_E = 8
_EP = 128
_BM = 512
_NBLK = _N // _BM
_BMC = 256
_NS = _N * 2 + _E * _BMC
_NT = _NS // _BMC
_NEG = -1e30

_NW = 32
_TPW = _N // _NW
_CH = 64
_NCH = _TPW // _CH

_MHI = -65536
_RND = 0x8000


def _route_block(x_ref, gw_ref, gb_ref, xb_ref, meta_ref, metaf_ref, pg_ref,
                 counts_ref, carry_ref):
    i = pl.program_id(0)

    @pl.when(i == 0)
    def _():
        carry_ref[...] = jnp.zeros((1, _E), jnp.float32)

    x = x_ref[...]
    bits = lax.bitcast_convert_type(x, jnp.int32)
    rb = bits + 0x7FFF + (lax.shift_right_logical(bits, 16) & 1)
    xb_ref[...] = (lax.shift_right_logical(rb[:, :_DW], 16)
                   | (rb[:, _DW:] & _MHI))
    logits = lax.dot_general(x, gw_ref[...], (((1,), (1,)), ((), ())),
                             preferred_element_type=jnp.float32) + gb_ref[...]
    eidx = lax.broadcasted_iota(jnp.int32, logits.shape, 1)
    m1 = jnp.max(logits, axis=1, keepdims=True)
    a1 = jnp.min(jnp.where(logits >= m1, eidx, _EP), axis=1, keepdims=True)
    mask1 = eidx == a1
    l2 = jnp.where(mask1, _NEG, logits)
    m2 = jnp.max(l2, axis=1, keepdims=True)
    a2 = jnp.min(jnp.where(l2 >= m2, eidx, _EP), axis=1, keepdims=True)
    mask2 = eidx == a2
    w1 = jax.nn.sigmoid(m1)
    w2 = jax.nn.sigmoid(m2)

    m1e = mask1[:, :_E]
    m2e = mask2[:, :_E]
    cnt = m1e.astype(jnp.float32) + m2e.astype(jnp.float32)
    r_io = lax.broadcasted_iota(jnp.int32, (_BM, _BM), 0)
    c_io = lax.broadcasted_iota(jnp.int32, (_BM, _BM), 1)
    ltri = (r_io > c_io).astype(jnp.bfloat16)
    excl = lax.dot_general(ltri, cnt.astype(jnp.bfloat16),
                           (((1,), (0,)), ((), ())),
                           preferred_element_type=jnp.float32)
    rankf = excl + carry_ref[...]
    r1 = jnp.sum(jnp.where(m1e, rankf, 0.0), axis=1, keepdims=True)
    r2 = jnp.sum(jnp.where(m2e, rankf, 0.0), axis=1, keepdims=True)
    carry_ref[...] = carry_ref[...] + jnp.sum(cnt, axis=0, keepdims=True)

    zi = jnp.zeros((_BM, 2), jnp.int32)
    meta_ref[...] = jnp.concatenate(
        [a1, a2, r1.astype(jnp.int32), r2.astype(jnp.int32), zi, zi], axis=1)
    zf = jnp.zeros((_BM, 2), jnp.float32)
    metaf_ref[...] = jnp.concatenate([w1, w2, zf, zf, zf], axis=1)

    @pl.when(i == _NBLK - 1)
    def _():
        total = carry_ref[...]
        aligned = jnp.ceil(total / _BMC) * _BMC
        e_r = lax.broadcasted_iota(jnp.int32, (_E, _E), 0)
        e_c = lax.broadcasted_iota(jnp.int32, (_E, _E), 1)
        before = (e_r < e_c).astype(jnp.float32)
        pg = lax.dot_general(aligned, before, (((1,), (0,)), ((), ())),
                             preferred_element_type=jnp.float32)
        pg_ref[...] = jnp.concatenate(
            [pg.astype(jnp.int32), jnp.zeros((1, 16 - _E), jnp.int32)],
            axis=1)
        counts_ref[...] = total.astype(jnp.int32)


def _route(x, gw_p, gb_p):
    return pl.pallas_call(
        _route_block,
        grid=(_NBLK,),
        in_specs=[
            pl.BlockSpec((_BM, _D), lambda i: (i, 0)),
            pl.BlockSpec((_EP, _D), lambda i: (0, 0)),
            pl.BlockSpec((1, _EP), lambda i: (0, 0)),
        ],
        out_specs=[
            pl.BlockSpec((_BM, _DW), lambda i: (i, 0)),
            pl.BlockSpec((_BM, 8), lambda i: (i, 0)),
            pl.BlockSpec((_BM, 8), lambda i: (i, 0)),
            pl.BlockSpec((1, 16), lambda i: (0, 0)),
            pl.BlockSpec((1, _E), lambda i: (0, 0)),
        ],
        out_shape=[
            jax.ShapeDtypeStruct((_N, _DW), jnp.int32),
            jax.ShapeDtypeStruct((_N, 8), jnp.int32),
            jax.ShapeDtypeStruct((_N, 8), jnp.float32),
            jax.ShapeDtypeStruct((1, 16), jnp.int32),
            jax.ShapeDtypeStruct((1, _E), jnp.int32),
        ],
        scratch_shapes=[pltpu.VMEM((1, _E), jnp.float32)],
        compiler_params=pltpu.CompilerParams(
            dimension_semantics=("arbitrary",),
        ),
    )(x, gw_p, gb_p)


def _pos_block(meta_ref, pg_ref, pos1_ref, pos2_ref):
    m = meta_ref[...]
    pg = pg_ref[...]
    ei = lax.broadcasted_iota(jnp.int32, (_N, 16), 1)
    s1 = jnp.sum(jnp.where(ei == m[:, 0:1], pg, 0), axis=1)
    s2 = jnp.sum(jnp.where(ei == m[:, 1:2], pg, 0), axis=1)
    pos1_ref[...] = s1 + m[:, 2]
    pos2_ref[...] = s2 + m[:, 3]


def _positions(meta, pg16):
    return pl.pallas_call(
        _pos_block,
        out_shape=[
            jax.ShapeDtypeStruct((_N,), jnp.int32),
            jax.ShapeDtypeStruct((_N,), jnp.int32),
        ],
    )(meta, pg16)


def _sc_wid():
    return lax.axis_index("s") * 2 + lax.axis_index("c")


@functools.cache
def _build_dispatch():
    mesh = plsc.VectorSubcoreMesh(core_axis_name="c", subcore_axis_name="s")

    @functools.partial(
        pl.kernel, mesh=mesh,
        out_type=jax.ShapeDtypeStruct((_NS, _DW), jnp.int32),
        scratch_types=[
            pltpu.VMEM((_NCH, _CH), jnp.int32),
            pltpu.VMEM((_NCH, _CH), jnp.int32),
            pltpu.VMEM((_CH, _DW), jnp.int32),
            pltpu.VMEM((_CH, _DW), jnp.int32),
            pltpu.SemaphoreType.DMA,
            pltpu.SemaphoreType.DMA,
            pltpu.SemaphoreType.DMA,
            pltpu.SemaphoreType.DMA,
            pltpu.SemaphoreType.DMA,
            pltpu.SemaphoreType.DMA,
        ],
    )
    def _dispatch(xb_hbm, pos1_hbm, pos2_hbm, xs_hbm,
                  p1a, p2a, v0, v1, l0, l1, sa0, sb0, sa1, sb1):
        base = _sc_wid() * _TPW
        row0 = _sc_wid() * _NCH
        pltpu.sync_copy(pos1_hbm.at[pl.ds(row0, _NCH)], p1a)
        pltpu.sync_copy(pos2_hbm.at[pl.ds(row0, _NCH)], p2a)
        bufs = (v0, v1)
        lsems = (l0, l1)
        ssems = ((sa0, sb0), (sa1, sb1))
        loads = [None] * _NCH
        scat = [None] * _NCH

        def start_load(ci):
            loads[ci] = pltpu.async_copy(
                xb_hbm.at[pl.ds(base + ci * _CH, _CH)], bufs[ci % 2],
                lsems[ci % 2])

        start_load(0)
        for ci in range(_NCH):
            if ci >= 1:
                ca, cb = scat[ci - 1]
                ca.wait()
                cb.wait()
            if ci + 1 < _NCH:
                start_load(ci + 1)
            loads[ci].wait()
            sa, sb = ssems[ci % 2]
            ca = pltpu.async_copy(bufs[ci % 2], xs_hbm.at[p1a.at[ci]], sa)
            cb = pltpu.async_copy(bufs[ci % 2], xs_hbm.at[p2a.at[ci]], sb)
            scat[ci] = (ca, cb)
        ca, cb = scat[_NCH - 1]
        ca.wait()
        cb.wait()

    return _dispatch


def _unpack_f32(w):
    lo = lax.bitcast_convert_type(lax.shift_left(w, 16), jnp.float32)
    hi = lax.bitcast_convert_type(w & _MHI, jnp.float32)
    return jnp.concatenate([lo, hi], axis=1)


def _gmm_block(te_ref, xs_ref, w_ref, b_ref, ys_ref):
    xb = _unpack_f32(xs_ref[...]).astype(jnp.bfloat16)
    acc = lax.dot_general(xb, w_ref[0], (((1,), (1,)), ((), ())),
                          preferred_element_type=jnp.float32)
    acc = acc + b_ref[0]
    bits = lax.bitcast_convert_type(acc, jnp.int32)
    rb = bits + 0x7FFF + (lax.shift_right_logical(bits, 16) & 1)
    ys_ref[...] = (lax.shift_right_logical(rb[:, :_DW], 16)
                   | (rb[:, _DW:] & _MHI))


def _gmm(xs_i32, w_bf16, b3, tile_expert):
    return pl.pallas_call(
        _gmm_block,
        grid_spec=pltpu.PrefetchScalarGridSpec(
            num_scalar_prefetch=1,
            grid=(_NT,),
            in_specs=[
                pl.BlockSpec((_BMC, _DW), lambda i, te: (i, 0)),
                pl.BlockSpec((1, _D, _D), lambda i, te: (te[i], 0, 0)),
                pl.BlockSpec((1, 1, _D), lambda i, te: (te[i], 0, 0)),
            ],
            out_specs=pl.BlockSpec((_BMC, _DW), lambda i, te: (i, 0)),
        ),
        out_shape=jax.ShapeDtypeStruct((_NS, _DW), jnp.int32),
        compiler_params=pltpu.CompilerParams(
            dimension_semantics=("arbitrary",),
        ),
    )(tile_expert, xs_i32, w_bf16, b3)


def _comb_block(ya_ref, yb_ref, mf_ref, o_ref):
    mf = mf_ref[...]
    o_ref[...] = (mf[:, 0:1] * _unpack_f32(ya_ref[...])
                  + mf[:, 1:2] * _unpack_f32(yb_ref[...]))


def _comb(ya, yb, metaf):
    return pl.pallas_call(
        _comb_block,
        grid=(_NBLK,),
        in_specs=[
            pl.BlockSpec((_BM, _DW), lambda i: (i, 0)),
            pl.BlockSpec((_BM, _DW), lambda i: (i, 0)),
            pl.BlockSpec((_BM, 8), lambda i: (i, 0)),
        ],
        out_specs=pl.BlockSpec((_BM, _D), lambda i: (i, 0)),
        out_shape=jax.ShapeDtypeStruct((_N, _D), jnp.float32),
        compiler_params=pltpu.CompilerParams(
            dimension_semantics=("arbitrary",),
        ),
    )(ya, yb, metaf)


@functools.cache
def _build_combine():
    mesh = plsc.VectorSubcoreMesh(core_axis_name="c", subcore_axis_name="s")

    @functools.partial(
        pl.kernel, mesh=mesh,
        out_type=[
            jax.ShapeDtypeStruct((_N, _DW), jnp.int32),
            jax.ShapeDtypeStruct((_N, _DW), jnp.int32),
        ],
        scratch_types=[
            pltpu.VMEM((_NCH, _CH), jnp.int32),
            pltpu.VMEM((_NCH, _CH), jnp.int32),
            pltpu.VMEM((_CH, _DW), jnp.int32),
            pltpu.VMEM((_CH, _DW), jnp.int32),
            pltpu.VMEM((_CH, _DW), jnp.int32),
            pltpu.VMEM((_CH, _DW), jnp.int32),
            pltpu.SemaphoreType.DMA,
            pltpu.SemaphoreType.DMA,
            pltpu.SemaphoreType.DMA,
            pltpu.SemaphoreType.DMA,
            pltpu.SemaphoreType.DMA,
            pltpu.SemaphoreType.DMA,
            pltpu.SemaphoreType.DMA,
            pltpu.SemaphoreType.DMA,
        ],
    )
    def _combine(ys_hbm, pos1_hbm, pos2_hbm, ya_hbm, yb_hbm,
                 p1a, p2a, a0, b0, a1, b1,
                 ga0, gb0, ga1, gb1, sa0, sb0, sa1, sb1):
        base = _sc_wid() * _TPW
        row0 = _sc_wid() * _NCH
        pltpu.sync_copy(pos1_hbm.at[pl.ds(row0, _NCH)], p1a)
        pltpu.sync_copy(pos2_hbm.at[pl.ds(row0, _NCH)], p2a)
        abufs = (a0, a1)
        bbufs = (b0, b1)
        gsems = ((ga0, gb0), (ga1, gb1))
        ssems = ((sa0, sb0), (sa1, sb1))
        gath = [None] * _NCH
        stor = [None] * _NCH

        def start_gather(ci):
            ga, gb = gsems[ci % 2]
            gath[ci] = (
                pltpu.async_copy(ys_hbm.at[p1a.at[ci]], abufs[ci % 2], ga),
                pltpu.async_copy(ys_hbm.at[p2a.at[ci]], bbufs[ci % 2], gb),
            )

        start_gather(0)
        for ci in range(_NCH):
            t0 = base + ci * _CH
            if ci >= 1:
                sa, sb = stor[ci - 1]
                sa.wait()
                sb.wait()
            if ci + 1 < _NCH:
                start_gather(ci + 1)
            g1, g2 = gath[ci]
            g1.wait()
            g2.wait()
            sa, sb = ssems[ci % 2]
            stor[ci] = (
                pltpu.async_copy(abufs[ci % 2], ya_hbm.at[pl.ds(t0, _CH)], sa),
                pltpu.async_copy(bbufs[ci % 2], yb_hbm.at[pl.ds(t0, _CH)], sb),
            )
        sa, sb = stor[_NCH - 1]
        sa.wait()
        sb.wait()

    return _combine


def kernel(inputs, gate_W, gate_b, expert_W, expert_b):
    x = inputs.reshape(_N, _D)
    gw_p = jnp.zeros((_EP, _D), jnp.float32).at[:_E].set(gate_W)
    gb_p = jnp.full((1, _EP), _NEG, jnp.float32).at[0, :_E].set(gate_b)
    w_bf16 = expert_W.astype(jnp.bfloat16)
    b3 = expert_b.reshape(_E, 1, _D)

    xb_i32, meta, metaf, pg16, counts = _route(x, gw_p, gb_p)
    pos1, pos2 = _positions(meta, pg16)

    aligned = (counts[0] + (_BMC - 1)) // _BMC * _BMC
    ends = jnp.cumsum(aligned)
    tile_starts = jnp.arange(_NT, dtype=jnp.int32) * _BMC
    tile_expert = jnp.sum(
        (tile_starts[:, None] >= ends[None, :_E - 1]).astype(jnp.int32),
        axis=1)

    pos1_2d = pos1.reshape(_N // _CH, _CH)
    pos2_2d = pos2.reshape(_N // _CH, _CH)

    xs_i32 = _build_dispatch()(xb_i32, pos1_2d, pos2_2d)
    ys_i32 = _gmm(xs_i32, w_bf16, b3, tile_expert)
    ya, yb = _build_combine()(ys_i32, pos1_2d, pos2_2d)
    return _comb(ya, yb, metaf).reshape(_B, _T, _D)


# speedup vs baseline: 5.3643x; 1.0178x over previous
import functools

import jax
import jax.numpy as jnp
from jax import lax
from jax.experimental import pallas as pl
from jax.experimental.pallas import tpu as pltpu
from jax.experimental.pallas import tpu_sc as plsc

_B, _T, _D = 4, 2048, 768
_DW = _D // 2
_N = _B * _T
_E = 8
_EP = 128
_BM = 512
_NBLK = _N // _BM
_BMA = 1024
_NBLKA = _N // _BMA
_BMC = 256
_NS = _N * 2 + _E * _BMC
_NT = _NS // _BMC
_NEG = -1e30

_NW = 32
_TPW = _N // _NW
_CH = 64
_NCH = _TPW // _CH

_MHI = -65536
_RND = 0x8000


def _route_block(x_ref, gw_ref, gb_ref, xb_ref, meta_ref, pg_ref,
                 carry_ref):
    i = pl.program_id(0)

    @pl.when(i == 0)
    def _():
        carry_ref[...] = jnp.zeros((1, _E), jnp.float32)

    x = x_ref[...]
    bits = lax.bitcast_convert_type(x, jnp.int32)
    rb = bits + 0x7FFF + (lax.shift_right_logical(bits, 16) & 1)
    xb_ref[...] = (lax.shift_right_logical(rb[:, :_DW], 16)
                   | (rb[:, _DW:] & _MHI))
    logits = lax.dot_general(x, gw_ref[...], (((1,), (1,)), ((), ())),
                             preferred_element_type=jnp.float32) + gb_ref[...]
    eidx = lax.broadcasted_iota(jnp.int32, logits.shape, 1)
    m1 = jnp.max(logits, axis=1, keepdims=True)
    a1 = jnp.min(jnp.where(logits >= m1, eidx, _EP), axis=1, keepdims=True)
    mask1 = eidx == a1
    l2 = jnp.where(mask1, _NEG, logits)
    m2 = jnp.max(l2, axis=1, keepdims=True)
    a2 = jnp.min(jnp.where(l2 >= m2, eidx, _EP), axis=1, keepdims=True)
    mask2 = eidx == a2
    w1 = jax.nn.sigmoid(m1)
    w2 = jax.nn.sigmoid(m2)

    m1e = mask1[:, :_E]
    m2e = mask2[:, :_E]
    cnt = m1e.astype(jnp.float32) + m2e.astype(jnp.float32)
    r_io = lax.broadcasted_iota(jnp.int32, (_BMA, _BMA), 0)
    c_io = lax.broadcasted_iota(jnp.int32, (_BMA, _BMA), 1)
    ltri = (r_io > c_io).astype(jnp.bfloat16)
    excl = lax.dot_general(ltri, cnt.astype(jnp.bfloat16),
                           (((1,), (0,)), ((), ())),
                           preferred_element_type=jnp.float32)
    rankf = excl + carry_ref[...]
    r1 = jnp.sum(jnp.where(m1e, rankf, 0.0), axis=1, keepdims=True)
    r2 = jnp.sum(jnp.where(m2e, rankf, 0.0), axis=1, keepdims=True)
    carry_ref[...] = carry_ref[...] + jnp.sum(cnt, axis=0, keepdims=True)

    zi = jnp.zeros((_BMA, 2), jnp.int32)
    meta_ref[...] = jnp.concatenate(
        [a1, a2, r1.astype(jnp.int32), r2.astype(jnp.int32),
         lax.bitcast_convert_type(w1, jnp.int32),
         lax.bitcast_convert_type(w2, jnp.int32), zi], axis=1)

    @pl.when(i == _NBLKA - 1)
    def _():
        total = carry_ref[...]
        aligned = jnp.ceil(total / _BMC) * _BMC
        e_r = lax.broadcasted_iota(jnp.int32, (_E, _E), 0)
        e_c = lax.broadcasted_iota(jnp.int32, (_E, _E), 1)
        before = (e_r < e_c).astype(jnp.float32)
        pg = lax.dot_general(aligned, before, (((1,), (0,)), ((), ())),
                             preferred_element_type=jnp.float32)
        pg_ref[...] = jnp.concatenate(
            [pg.astype(jnp.int32), jnp.zeros((1, 16 - _E), jnp.int32)],
            axis=1)


def _route(x, gw_p, gb_p):
    return pl.pallas_call(
        _route_block,
        grid=(_NBLKA,),
        in_specs=[
            pl.BlockSpec((_BMA, _D), lambda i: (i, 0)),
            pl.BlockSpec((_EP, _D), lambda i: (0, 0)),
            pl.BlockSpec((1, _EP), lambda i: (0, 0)),
        ],
        out_specs=[
            pl.BlockSpec((_BMA, _DW), lambda i: (i, 0)),
            pl.BlockSpec((_BMA, 8), lambda i: (i, 0)),
            pl.BlockSpec((1, 16), lambda i: (0, 0)),
        ],
        out_shape=[
            jax.ShapeDtypeStruct((_N, _DW), jnp.int32),
            jax.ShapeDtypeStruct((_N, 8), jnp.int32),
            jax.ShapeDtypeStruct((1, 16), jnp.int32),
        ],
        scratch_shapes=[pltpu.VMEM((1, _E), jnp.float32)],
        compiler_params=pltpu.CompilerParams(
            dimension_semantics=("arbitrary",),
        ),
    )(x, gw_p, gb_p)


def _pos_block(meta_ref, pg_ref, pos1_ref, pos2_ref):
    m = meta_ref[...]
    pg = pg_ref[...]
    ei = lax.broadcasted_iota(jnp.int32, (_N, 16), 1)
    s1 = jnp.sum(jnp.where(ei == m[:, 0:1], pg, 0), axis=1)
    s2 = jnp.sum(jnp.where(ei == m[:, 1:2], pg, 0), axis=1)
    pos1_ref[...] = s1 + m[:, 2]
    pos2_ref[...] = s2 + m[:, 3]


def _positions(meta, pg16):
    return pl.pallas_call(
        _pos_block,
        out_shape=[
            jax.ShapeDtypeStruct((_N,), jnp.int32),
            jax.ShapeDtypeStruct((_N,), jnp.int32),
        ],
    )(meta, pg16)


def _sc_wid():
    return lax.axis_index("s") * 2 + lax.axis_index("c")


@functools.cache
def _build_dispatch():
    mesh = plsc.VectorSubcoreMesh(core_axis_name="c", subcore_axis_name="s")

    @functools.partial(
        pl.kernel, mesh=mesh,
        out_type=jax.ShapeDtypeStruct((_NS, _DW), jnp.int32),
        scratch_types=[
            pltpu.VMEM((_NCH, _CH), jnp.int32),
            pltpu.VMEM((_NCH, _CH), jnp.int32),
            pltpu.VMEM((_CH, _DW), jnp.int32),
            pltpu.VMEM((_CH, _DW), jnp.int32),
            pltpu.SemaphoreType.DMA,
            pltpu.SemaphoreType.DMA,
            pltpu.SemaphoreType.DMA,
            pltpu.SemaphoreType.DMA,
            pltpu.SemaphoreType.DMA,
            pltpu.SemaphoreType.DMA,
        ],
    )
    def _dispatch(xb_hbm, pos1_hbm, pos2_hbm, xs_hbm,
                  p1a, p2a, v0, v1, l0, l1, sa0, sb0, sa1, sb1):
        base = _sc_wid() * _TPW
        row0 = _sc_wid() * _NCH
        pltpu.sync_copy(pos1_hbm.at[pl.ds(row0, _NCH)], p1a)
        pltpu.sync_copy(pos2_hbm.at[pl.ds(row0, _NCH)], p2a)
        bufs = (v0, v1)
        lsems = (l0, l1)
        ssems = ((sa0, sb0), (sa1, sb1))
        loads = [None] * _NCH
        scat = [None] * _NCH

        def start_load(ci):
            loads[ci] = pltpu.async_copy(
                xb_hbm.at[pl.ds(base + ci * _CH, _CH)], bufs[ci % 2],
                lsems[ci % 2])

        start_load(0)
        for ci in range(_NCH):
            if ci >= 1:
                ca, cb = scat[ci - 1]
                ca.wait()
                cb.wait()
            if ci + 1 < _NCH:
                start_load(ci + 1)
            loads[ci].wait()
            sa, sb = ssems[ci % 2]
            ca = pltpu.async_copy(bufs[ci % 2], xs_hbm.at[p1a.at[ci]], sa)
            cb = pltpu.async_copy(bufs[ci % 2], xs_hbm.at[p2a.at[ci]], sb)
            scat[ci] = (ca, cb)
        ca, cb = scat[_NCH - 1]
        ca.wait()
        cb.wait()

    return _dispatch


def _unpack_f32(w):
    lo = lax.bitcast_convert_type(lax.shift_left(w, 16), jnp.float32)
    hi = lax.bitcast_convert_type(w & _MHI, jnp.float32)
    return jnp.concatenate([lo, hi], axis=1)


def _gmm_block(te_ref, xs_ref, w_ref, b_ref, ys_ref):
    xb = _unpack_f32(xs_ref[...]).astype(jnp.bfloat16)
    acc = lax.dot_general(xb, w_ref[0], (((1,), (1,)), ((), ())),
                          preferred_element_type=jnp.float32)
    acc = acc + b_ref[0]
    bits = lax.bitcast_convert_type(acc, jnp.int32)
    rb = bits + 0x7FFF + (lax.shift_right_logical(bits, 16) & 1)
    ys_ref[...] = (lax.shift_right_logical(rb[:, :_DW], 16)
                   | (rb[:, _DW:] & _MHI))


def _gmm(xs_i32, w_bf16, b3, tile_expert):
    return pl.pallas_call(
        _gmm_block,
        grid_spec=pltpu.PrefetchScalarGridSpec(
            num_scalar_prefetch=1,
            grid=(_NT,),
            in_specs=[
                pl.BlockSpec((_BMC, _DW), lambda i, te: (i, 0)),
                pl.BlockSpec((1, _D, _D), lambda i, te: (te[i], 0, 0)),
                pl.BlockSpec((1, 1, _D), lambda i, te: (te[i], 0, 0)),
            ],
            out_specs=pl.BlockSpec((_BMC, _DW), lambda i, te: (i, 0)),
        ),
        out_shape=jax.ShapeDtypeStruct((_NS, _DW), jnp.int32),
        compiler_params=pltpu.CompilerParams(
            dimension_semantics=("arbitrary",),
        ),
    )(tile_expert, xs_i32, w_bf16, b3)


def _comb_block(ya_ref, yb_ref, mf_ref, o_ref):
    m = mf_ref[...]
    w1 = lax.bitcast_convert_type(m[:, 4:5], jnp.float32)
    w2 = lax.bitcast_convert_type(m[:, 5:6], jnp.float32)
    o_ref[...] = (w1 * _unpack_f32(ya_ref[...])
                  + w2 * _unpack_f32(yb_ref[...]))


def _comb(ya, yb, metaf):
    return pl.pallas_call(
        _comb_block,
        grid=(_NBLK,),
        in_specs=[
            pl.BlockSpec((_BM, _DW), lambda i: (i, 0)),
            pl.BlockSpec((_BM, _DW), lambda i: (i, 0)),
            pl.BlockSpec((_BM, 8), lambda i: (i, 0)),
        ],
        out_specs=pl.BlockSpec((_BM, _D), lambda i: (i, 0)),
        out_shape=jax.ShapeDtypeStruct((_N, _D), jnp.float32),
        compiler_params=pltpu.CompilerParams(
            dimension_semantics=("arbitrary",),
        ),
    )(ya, yb, metaf)


@functools.cache
def _build_combine():
    mesh = plsc.VectorSubcoreMesh(core_axis_name="c", subcore_axis_name="s")

    @functools.partial(
        pl.kernel, mesh=mesh,
        out_type=[
            jax.ShapeDtypeStruct((_N, _DW), jnp.int32),
            jax.ShapeDtypeStruct((_N, _DW), jnp.int32),
        ],
        scratch_types=[
            pltpu.VMEM((_NCH, _CH), jnp.int32),
            pltpu.VMEM((_NCH, _CH), jnp.int32),
            pltpu.VMEM((_CH, _DW), jnp.int32),
            pltpu.VMEM((_CH, _DW), jnp.int32),
            pltpu.VMEM((_CH, _DW), jnp.int32),
            pltpu.VMEM((_CH, _DW), jnp.int32),
            pltpu.SemaphoreType.DMA,
            pltpu.SemaphoreType.DMA,
            pltpu.SemaphoreType.DMA,
            pltpu.SemaphoreType.DMA,
            pltpu.SemaphoreType.DMA,
            pltpu.SemaphoreType.DMA,
            pltpu.SemaphoreType.DMA,
            pltpu.SemaphoreType.DMA,
        ],
    )
    def _combine(ys_hbm, pos1_hbm, pos2_hbm, ya_hbm, yb_hbm,
                 p1a, p2a, a0, b0, a1, b1,
                 ga0, gb0, ga1, gb1, sa0, sb0, sa1, sb1):
        base = _sc_wid() * _TPW
        row0 = _sc_wid() * _NCH
        pltpu.sync_copy(pos1_hbm.at[pl.ds(row0, _NCH)], p1a)
        pltpu.sync_copy(pos2_hbm.at[pl.ds(row0, _NCH)], p2a)
        abufs = (a0, a1)
        bbufs = (b0, b1)
        gsems = ((ga0, gb0), (ga1, gb1))
        ssems = ((sa0, sb0), (sa1, sb1))
        gath = [None] * _NCH
        stor = [None] * _NCH

        def start_gather(ci):
            ga, gb = gsems[ci % 2]
            gath[ci] = (
                pltpu.async_copy(ys_hbm.at[p1a.at[ci]], abufs[ci % 2], ga),
                pltpu.async_copy(ys_hbm.at[p2a.at[ci]], bbufs[ci % 2], gb),
            )

        start_gather(0)
        for ci in range(_NCH):
            t0 = base + ci * _CH
            if ci >= 1:
                sa, sb = stor[ci - 1]
                sa.wait()
                sb.wait()
            if ci + 1 < _NCH:
                start_gather(ci + 1)
            g1, g2 = gath[ci]
            g1.wait()
            g2.wait()
            sa, sb = ssems[ci % 2]
            stor[ci] = (
                pltpu.async_copy(abufs[ci % 2], ya_hbm.at[pl.ds(t0, _CH)], sa),
                pltpu.async_copy(bbufs[ci % 2], yb_hbm.at[pl.ds(t0, _CH)], sb),
            )
        sa, sb = stor[_NCH - 1]
        sa.wait()
        sb.wait()

    return _combine


def kernel(inputs, gate_W, gate_b, expert_W, expert_b):
    x = inputs.reshape(_N, _D)
    gw_p = jnp.zeros((_EP, _D), jnp.float32).at[:_E].set(gate_W)
    gb_p = jnp.full((1, _EP), _NEG, jnp.float32).at[0, :_E].set(gate_b)
    w_bf16 = expert_W.astype(jnp.bfloat16)
    b3 = expert_b.reshape(_E, 1, _D)

    xb_i32, meta, pg16 = _route(x, gw_p, gb_p)
    pos1, pos2 = _positions(meta, pg16)

    ends7 = pg16[0, 1:_E]
    tile_starts = jnp.arange(_NT, dtype=jnp.int32) * _BMC
    tile_expert = jnp.sum(
        (tile_starts[:, None] >= ends7[None, :]).astype(jnp.int32),
        axis=1)

    pos1_2d = pos1.reshape(_N // _CH, _CH)
    pos2_2d = pos2.reshape(_N // _CH, _CH)

    xs_i32 = _build_dispatch()(xb_i32, pos1_2d, pos2_2d)
    ys_i32 = _gmm(xs_i32, w_bf16, b3, tile_expert)
    ya, yb = _build_combine()(ys_i32, pos1_2d, pos2_2d)
    return _comb(ya, yb, meta).reshape(_B, _T, _D)
